# Initial kernel scaffold; baseline (speedup 1.0000x reference)
import jax
import jax.numpy as jnp
from jax.experimental import pallas as pl


def kernel(center_words, positive_context, negative_context, input_emb, output_emb):
    raise NotImplementedError("write your pallas kernel here")


# speedup vs baseline: 5.0544x; 5.0544x over previous
import functools

import jax
import jax.numpy as jnp
from jax import lax
from jax.experimental import pallas as pl
from jax.experimental.pallas import tpu as pltpu
from jax.experimental.pallas import tpu_sc as plsc

NC = 2
NS = 16
NW = NC * NS
CHUNK = 128


def _make_sc_scores(V, D, B, NCTX):
    S = B // NW
    KC = S // CHUNK
    mesh = plsc.VectorSubcoreMesh(core_axis_name="c", subcore_axis_name="s")

    def fire(emb, idx, rows, sem):
        for k in range(KC):
            pltpu.async_copy(emb.at[idx.at[k]], rows.at[pl.ds(k * CHUNK, CHUNK)], sem)

    def drain(emb, idx, rows, sem):
        for k in range(KC):
            pltpu.make_async_copy(
                emb.at[idx.at[k]], rows.at[pl.ds(k * CHUNK, CHUNK)], sem
            ).wait()

    @functools.partial(
        pl.kernel,
        out_type=jax.ShapeDtypeStruct((NCTX * B,), jnp.float32),
        mesh=mesh,
        compiler_params=pltpu.CompilerParams(
            needs_layout_passes=False, use_tc_tiling_on_sc=False
        ),
        scratch_types=[
            pltpu.VMEM((KC, CHUNK), jnp.int32),
            pltpu.VMEM((KC, CHUNK), jnp.int32),
            pltpu.VMEM((KC, CHUNK), jnp.int32),
            pltpu.VMEM((S, D), jnp.float32),
            pltpu.VMEM((S, D), jnp.float32),
            pltpu.VMEM((S, D), jnp.float32),
            pltpu.VMEM((16 * 17,), jnp.float32),
            pltpu.VMEM((S,), jnp.float32),
            pltpu.SemaphoreType.DMA,
            pltpu.SemaphoreType.DMA,
            pltpu.SemaphoreType.DMA,
        ],
    )
    def sc_scores(cw_hbm, ctx_hbm, in_emb, out_emb, out_hbm,
                  cidx, xidx0, xidx1, crow, xrow0, xrow1, tmp, srow,
                  csem, sem0, sem1):
        wid = lax.axis_index("s") * NC + lax.axis_index("c")
        wbase = wid * S

        rid17 = lax.iota(jnp.int32, 16) * 17

        def compute_chunk(xrow, j):
            @pl.loop(0, S // 16)
            def _(g):
                b0 = g * 16
                for e in range(16):
                    b = b0 + e
                    v = crow[b, pl.ds(0, 16)] * xrow[b, pl.ds(0, 16)]
                    for q in range(1, D // 16):
                        v = v + crow[b, pl.ds(q * 16, 16)] * xrow[b, pl.ds(q * 16, 16)]
                    tmp[pl.ds(e * 17, 16)] = v
                acc = plsc.load_gather(tmp, [rid17])
                for c in range(1, 16):
                    acc = acc + plsc.load_gather(tmp, [rid17 + c])
                srow[pl.ds(b0, 16)] = acc
            off = pl.multiple_of(j * B + wbase, S)
            pltpu.sync_copy(srow, out_hbm.at[pl.ds(off, S)])

        pltpu.sync_copy(cw_hbm.at[wid], cidx)
        fire(in_emb, cidx, crow, csem)
        pltpu.sync_copy(ctx_hbm.at[0, wid], xidx0)
        fire(out_emb, xidx0, xrow0, sem0)
        drain(in_emb, cidx, crow, csem)

        @pl.loop(0, NCTX - 1, step=2)
        def _(j):
            pltpu.sync_copy(ctx_hbm.at[j + 1, wid], xidx1)
            fire(out_emb, xidx1, xrow1, sem1)
            drain(out_emb, xidx0, xrow0, sem0)
            compute_chunk(xrow0, j)
            pltpu.sync_copy(ctx_hbm.at[j + 2, wid], xidx0)
            fire(out_emb, xidx0, xrow0, sem0)
            drain(out_emb, xidx1, xrow1, sem1)
            compute_chunk(xrow1, j + 1)

        drain(out_emb, xidx0, xrow0, sem0)
        compute_chunk(xrow0, NCTX - 1)

    return sc_scores


def _make_tc_loss(B, NEG):
    NCTX = NEG + 1

    def body(s_ref, o_ref):
        s = s_ref[...]
        row = lax.broadcasted_iota(jnp.int32, s.shape, 0)
        x = jnp.where(row == 0, s, -s)
        ls = jax.nn.log_sigmoid(x)
        w = jnp.where(row == 0, 1.0 / B, 1.0 / (B * NEG))
        o_ref[0, 0] = -jnp.sum(ls * w)

    return pl.pallas_call(
        body,
        out_shape=jax.ShapeDtypeStruct((1, 1), jnp.float32),
        out_specs=pl.BlockSpec(memory_space=pltpu.SMEM),
    )


def kernel(center_words, positive_context, negative_context, input_emb, output_emb):
    B = center_words.shape[0]
    NEG = negative_context.shape[1]
    V, D = input_emb.shape
    NCTX = NEG + 1

    S = B // NW
    cw = center_words.astype(jnp.int32).reshape(NW, S // CHUNK, CHUNK)
    ctx = jnp.concatenate(
        [positive_context[None, :], negative_context.T], axis=0
    ).astype(jnp.int32).reshape(NCTX, NW, S // CHUNK, CHUNK)

    scores = _make_sc_scores(V, D, B, NCTX)(cw, ctx, input_emb, output_emb)
    loss = _make_tc_loss(B, NEG)(scores.reshape(NCTX, B))
    return loss[0, 0]
